# baseline (speedup 1.0000x reference)
import functools

import jax
import jax.numpy as jnp
from jax import lax
from jax.experimental import pallas as pl
from jax.experimental.pallas import tpu as pltpu
from jax.experimental.pallas import tpu_sc as plsc

N = 10000
E = 160000
D = 128
L = 3
NET = 2
C = 3
T_SRCS = (0, 1)
T_TGTS = (1, 0)

NC, NS, LN = 2, 16, 16
NW = NC * NS
NPT = 320
NP = NW * NPT
CE = 4000
NCH = E // CE
CM = 64
CEP = CE + CM
NEG = -1e30

mesh = plsc.VectorSubcoreMesh(core_axis_name="c", subcore_axis_name="s")


@functools.partial(
    pl.kernel, mesh=mesh,
    out_type=(
        jax.ShapeDtypeStruct((NET * NW * NCH * CEP,), jnp.int32),
        jax.ShapeDtypeStruct((NET * NW * NCH * CEP,), jnp.int32),
        jax.ShapeDtypeStruct((NET * NW * NCH * LN,), jnp.int32),
        jax.ShapeDtypeStruct((NET * NP,), jnp.float32),
    ),
    scratch_types=[
        pltpu.VMEM((CE,), jnp.int32),
        pltpu.VMEM((CE,), jnp.int32),
        pltpu.VMEM((CEP,), jnp.int32),
        pltpu.VMEM((CEP,), jnp.int32),
        pltpu.VMEM((NPT,), jnp.float32),
        pltpu.VMEM((NCH * LN,), jnp.int32),
    ],
    compiler_params=pltpu.CompilerParams(needs_layout_passes=False),
)
def prep_kernel(ess_hbm, bsrc_hbm, bdst_hbm, cnt_hbm, hs_hbm,
                srcv, dstv, ssrc, sdst, hsv, cntv):
    wid = lax.axis_index("s") * NC + lax.axis_index("c")
    lo = wid * NPT

    for j in range(NET):
        def init_hs(g, _):
            hsv[pl.ds(g * LN, LN)] = jnp.zeros((LN,), jnp.float32)
            return 0
        lax.fori_loop(0, NPT // LN, init_hs, 0)

        def chunk_body(k, _):
            pltpu.sync_copy(ess_hbm.at[pl.ds((2 * j + 0) * E + k * CE, CE)], srcv)
            pltpu.sync_copy(ess_hbm.at[pl.ds((2 * j + 1) * E + k * CE, CE)], dstv)

            def prefill(g, _):
                ssrc[pl.ds(g * LN, LN)] = jnp.zeros((LN,), jnp.int32)
                sdst[pl.ds(g * LN, LN)] = jnp.full((LN,), NPT, jnp.int32)
                return 0
            lax.fori_loop(0, CEP // LN, prefill, 0)

            def scan_body(g, ptr):
                s16 = srcv[pl.ds(g * LN, LN)]
                d16 = dstv[pl.ds(g * LN, LN)]
                one16 = jnp.ones((LN,), jnp.int32)
                zero16 = jnp.zeros((LN,), jnp.int32)
                inr = (d16 >= lo) & (d16 < lo + NPT)
                nonself = s16 != d16
                m1 = inr & nonself
                mi = jnp.where(m1, one16, zero16)
                inc = plsc.cumsum(mi)
                pos = ptr + inc - mi
                plsc.store_scatter(ssrc, [pos], s16, mask=m1)
                plsc.store_scatter(sdst, [pos], d16 - lo, mask=m1)
                ms = inr & jnp.logical_not(nonself)
                plsc.store_scatter(hsv, [d16 - lo], jnp.ones((LN,), jnp.float32),
                                   mask=ms)
                return ptr + inc[LN - 1]
            cnt = lax.fori_loop(0, CE // LN, scan_body, jnp.int32(0))

            cntv[pl.ds(k * LN, LN)] = jnp.full((LN,), cnt, jnp.int32)

            base = ((j * NW + wid) * NCH + k) * CEP
            pltpu.sync_copy(ssrc, bsrc_hbm.at[pl.ds(base, CEP)])
            pltpu.sync_copy(sdst, bdst_hbm.at[pl.ds(base, CEP)])
            return 0
        lax.fori_loop(0, NCH, chunk_body, 0)

        pltpu.sync_copy(cntv, cnt_hbm.at[pl.ds((j * NW + wid) * NCH * LN, NCH * LN)])
        pltpu.sync_copy(hsv, hs_hbm.at[pl.ds(j * NP + lo, NPT)])


@functools.partial(
    pl.kernel, mesh=mesh,
    out_type=jax.ShapeDtypeStruct((NET, NP, D), jnp.float32),
    scratch_types=[
        pltpu.VMEM((NPT + 1, D), jnp.float32),
        pltpu.VMEM((CM, D), jnp.float32),
        pltpu.VMEM((CM,), jnp.int32),
        pltpu.VMEM((CM,), jnp.int32),
        pltpu.VMEM((NCH * LN,), jnp.int32),
        pltpu.SemaphoreType.DMA,
    ],
    compiler_params=pltpu.CompilerParams(needs_layout_passes=False),
)
def segmax_kernel(v0_hbm, v1_hbm, bsrc_hbm, bdst_hbm, cnt_hbm, m_hbm,
                  accv, rows, idxv, dstlv, cntv, sem):
    wid = lax.axis_index("s") * NC + lax.axis_index("c")

    for j in range(NET):
        v_hbm = v0_hbm if j == 0 else v1_hbm

        def init_acc(r, _):
            for c in range(D // LN):
                accv[r, pl.ds(c * LN, LN)] = jnp.full((LN,), NEG, jnp.float32)
            return 0
        lax.fori_loop(0, NPT + 1, init_acc, 0)

        pltpu.sync_copy(cnt_hbm.at[pl.ds((j * NW + wid) * NCH * LN, NCH * LN)],
                        cntv)

        def chunk_body(k, _):
            cnt = cntv[pl.ds(k * LN, LN)][0]
            nsub = (cnt + (CM - 1)) // CM
            kbase = ((j * NW + wid) * NCH + k) * CEP

            def sub_body(s, _):
                pltpu.sync_copy(bsrc_hbm.at[pl.ds(kbase + s * CM, CM)], idxv)
                pltpu.sync_copy(bdst_hbm.at[pl.ds(kbase + s * CM, CM)], dstlv)
                pltpu.async_copy(v_hbm.at[idxv], rows, sem).wait()

                def edge_body(g, _):
                    dl16 = dstlv[pl.ds(g * LN, LN)]
                    for i in range(LN):
                        dl = dl16[i]
                        r = g * LN + i
                        for c in range(D // LN):
                            a = accv[dl, pl.ds(c * LN, LN)]
                            b = rows[r, pl.ds(c * LN, LN)]
                            accv[dl, pl.ds(c * LN, LN)] = jnp.maximum(a, b)
                    return 0
                lax.fori_loop(0, CM // LN, edge_body, 0)
                return 0
            lax.fori_loop(0, nsub, sub_body, 0)
            return 0
        lax.fori_loop(0, NCH, chunk_body, 0)

        pltpu.sync_copy(accv.at[pl.ds(0, NPT)],
                        m_hbm.at[j, pl.ds(wid * NPT, NPT)])


NB = 10
BN = N // NB


def _mm_body(x_ref, w_ref, b_ref, v_ref, u_ref):
    o = jnp.dot(x_ref[0], w_ref[0], preferred_element_type=jnp.float32)
    o = o + b_ref[0]
    v_ref[0] = o[:, :D]
    u_ref[0] = o[:, D:]


def _matmul(x2, wcat, bcat):
    return pl.pallas_call(
        _mm_body,
        grid=(2, NB),
        in_specs=[
            pl.BlockSpec((1, BN, D), lambda t, n: (t, n, 0)),
            pl.BlockSpec((1, D, 2 * D), lambda t, n: (t, 0, 0)),
            pl.BlockSpec((1, 1, 2 * D), lambda t, n: (t, 0, 0)),
        ],
        out_specs=[
            pl.BlockSpec((1, BN, D), lambda t, n: (t, n, 0)),
            pl.BlockSpec((1, BN, D), lambda t, n: (t, n, 0)),
        ],
        out_shape=[
            jax.ShapeDtypeStruct((2, N, D), jnp.float32),
            jax.ShapeDtypeStruct((2, N, D), jnp.float32),
        ],
    )(x2, wcat, bcat)


def _comb_math(u_ref, v_ref, m_ref, hs_ref, emb_ref):
    u = u_ref[0]
    v = v_ref[0]
    m = m_ref[0]
    hs = hs_ref[0]
    e0 = emb_ref[0, 0]
    e1 = emb_ref[0, 1]
    self_term = jnp.where(hs > 0.5, v + e1, NEG)
    return jax.nn.relu(u + jnp.maximum(m + e0, self_term))


def _combine_body(u_ref, v_ref, m_ref, hs_ref, emb_ref, o_ref):
    o_ref[0] = _comb_math(u_ref, v_ref, m_ref, hs_ref, emb_ref)


_COMB_SPECS = [
    pl.BlockSpec((1, BN, D), lambda t, n: (t, n, 0)),
    pl.BlockSpec((1, BN, D), lambda t, n: (1 - t, n, 0)),
    pl.BlockSpec((1, BN, D), lambda t, n: (1 - t, n, 0)),
    pl.BlockSpec((1, BN, 1), lambda t, n: (1 - t, n, 0)),
    pl.BlockSpec((1, 2, D), lambda t, n: (1 - t, 0, 0)),
]


def _combine(u, v, m, hs, emb_i):
    return pl.pallas_call(
        _combine_body,
        grid=(2, NB),
        in_specs=_COMB_SPECS,
        out_specs=pl.BlockSpec((1, BN, D), lambda t, n: (t, n, 0)),
        out_shape=jax.ShapeDtypeStruct((2, N, D), jnp.float32),
    )(u, v, m, hs, emb_i)


def _final_body(u_ref, v_ref, m_ref, hs_ref, emb_ref, wd_ref, bd_ref,
                last_ref, prob_ref):
    x = _comb_math(u_ref, v_ref, m_ref, hs_ref, emb_ref)
    o = jnp.dot(x, wd_ref[...], preferred_element_type=jnp.float32) + bd_ref[0]
    last_ref[0] = o
    z = o - jnp.max(o, axis=-1, keepdims=True)
    ez = jnp.exp(z)
    prob_ref[0] = ez / jnp.sum(ez, axis=-1, keepdims=True)


def _final(u, v, m, hs, emb_i, w_dec, b_dec):
    return pl.pallas_call(
        _final_body,
        grid=(2, NB),
        in_specs=_COMB_SPECS + [
            pl.BlockSpec((D, C), lambda t, n: (0, 0)),
            pl.BlockSpec((1, C), lambda t, n: (0, 0)),
        ],
        out_specs=[
            pl.BlockSpec((1, BN, C), lambda t, n: (t, n, 0)),
            pl.BlockSpec((1, BN, C), lambda t, n: (t, n, 0)),
        ],
        out_shape=[
            jax.ShapeDtypeStruct((2, N, C), jnp.float32),
            jax.ShapeDtypeStruct((2, N, C), jnp.float32),
        ],
    )(u, v, m, hs, emb_i, w_dec, b_dec)


def kernel(xs, ess, W_tgt, b_tgt, W_src, b_src, emb_se, W_dec, b_dec):
    bsrc, bdst, cnts, hs = prep_kernel(ess.reshape(-1))
    hs3 = hs.reshape(NET, NP, 1)

    x2 = xs
    for i in range(L):
        wcat = jnp.stack([
            jnp.concatenate([W_src[i, 0], W_tgt[i, 1]], axis=1),
            jnp.concatenate([W_src[i, 1], W_tgt[i, 0]], axis=1),
        ])
        bcat = jnp.stack([
            jnp.concatenate([b_src[i, 0], b_tgt[i, 1]])[None, :],
            jnp.concatenate([b_src[i, 1], b_tgt[i, 0]])[None, :],
        ])
        v2, u2 = _matmul(x2, wcat, bcat)
        m = segmax_kernel(v2[0], v2[1], bsrc, bdst, cnts)
        if i < L - 1:
            x2 = _combine(u2, v2, m, hs3, emb_se[i])
        else:
            last, probs = _final(u2, v2, m, hs3, emb_se[i],
                                 W_dec, b_dec[None, :])
    return (last, probs)
